# Initial kernel scaffold; baseline (speedup 1.0000x reference)
import jax
import jax.numpy as jnp
from jax.experimental import pallas as pl


def kernel(src, seg, prop_keys, prop_values, target_words, word_table, pos_table, seg_table, gamma, beta):
    raise NotImplementedError("write your pallas kernel here")


# speedup vs baseline: 1.0115x; 1.0115x over previous
import jax
import jax.numpy as jnp
from jax import lax
from jax.experimental import pallas as pl
from jax.experimental.pallas import tpu as pltpu
from jax.experimental.pallas import tpu_sc as plsc

EMB = 768
EPS = 1e-6
NW = 32
CHUNK = 32


def _sc_gather_body(idx_hbm, table_hbm, out_hbm, idx_v, rows_v, sem):
    wid = lax.axis_index("s") * 2 + lax.axis_index("c")
    per_w = out_hbm.shape[0] // NW
    base = wid * per_w

    def body(k, carry):
        off = base + k * CHUNK
        pltpu.sync_copy(idx_hbm.at[pl.ds(off, CHUNK)], idx_v)
        pltpu.async_copy(table_hbm.at[idx_v], rows_v, sem).wait()
        pltpu.sync_copy(rows_v, out_hbm.at[pl.ds(off, CHUNK)])
        return carry

    lax.fori_loop(0, per_w // CHUNK, body, 0)


def _sc_gather(w_idx, word_table):
    n = w_idx.shape[0]
    return pl.kernel(
        _sc_gather_body,
        out_type=jax.ShapeDtypeStruct((n, EMB), jnp.float32),
        mesh=plsc.VectorSubcoreMesh(core_axis_name="c", subcore_axis_name="s"),
        scratch_types=[
            pltpu.VMEM((CHUNK,), jnp.int32),
            pltpu.VMEM((CHUNK, EMB), jnp.float32),
            pltpu.SemaphoreType.DMA,
        ],
    )(w_idx, word_table)


def _ln(x, g, b):
    mean = jnp.mean(x, axis=-1, keepdims=True)
    xc = x - mean
    var = jnp.mean(xc * xc, axis=-1, keepdims=True)
    return xc * lax.rsqrt(var + EPS) * g + b


def _emb_body(x_ref, oh_ref, pos_ref, segt_ref, g_ref, b_ref, o_ref):
    seg_emb = jnp.dot(oh_ref[0], segt_ref[...], preferred_element_type=jnp.float32)
    x = x_ref[0] + pos_ref[...] + seg_emb
    o_ref[0] = _ln(x, g_ref[...], b_ref[...])


def _prop_body(x_ref, pos_ref, g_ref, b_ref, o_ref):
    x = x_ref[...] + pos_ref[...][None]
    o_ref[...] = _ln(x, g_ref[...][None], b_ref[...][None])


def _emb_ln(xg, seg_oh, pos_table, seg_pad, gamma2, beta2):
    return pl.pallas_call(
        _emb_body,
        grid=(32,),
        in_specs=[
            pl.BlockSpec((1, 512, EMB), lambda i: (i, 0, 0)),
            pl.BlockSpec((1, 512, 4), lambda i: (i, 0, 0)),
            pl.BlockSpec((512, EMB), lambda i: (0, 0)),
            pl.BlockSpec((4, EMB), lambda i: (0, 0)),
            pl.BlockSpec((1, EMB), lambda i: (0, 0)),
            pl.BlockSpec((1, EMB), lambda i: (0, 0)),
        ],
        out_specs=pl.BlockSpec((1, 512, EMB), lambda i: (i, 0, 0)),
        out_shape=jax.ShapeDtypeStruct((32, 512, EMB), jnp.float32),
    )(xg, seg_oh, pos_table, seg_pad, gamma2, beta2)


def _prop_ln(xp, pos_table, gamma2, beta2):
    n = xp.shape[0]
    g = 96
    return pl.pallas_call(
        _prop_body,
        grid=(n // g,),
        in_specs=[
            pl.BlockSpec((g, 8, EMB), lambda i: (i, 0, 0)),
            pl.BlockSpec((8, EMB), lambda i: (0, 0)),
            pl.BlockSpec((1, EMB), lambda i: (0, 0)),
            pl.BlockSpec((1, EMB), lambda i: (0, 0)),
        ],
        out_specs=pl.BlockSpec((g, 8, EMB), lambda i: (i, 0, 0)),
        out_shape=jax.ShapeDtypeStruct((n, 8, EMB), jnp.float32),
    )(xp, pos_table, gamma2, beta2)


def kernel(src, seg, prop_keys, prop_values, target_words,
           word_table, pos_table, seg_table, gamma, beta):
    b, l = src.shape
    _, t, k = prop_keys.shape
    n_src = b * l
    n_prop = b * t * k

    w_idx = jnp.concatenate([
        src.reshape(-1), prop_keys.reshape(-1),
        prop_values.reshape(-1), target_words.reshape(-1),
    ]).astype(jnp.int32)

    gathered = _sc_gather(w_idx, word_table)

    seg_oh = jax.nn.one_hot(seg, 4, dtype=jnp.float32)
    seg_pad = jnp.zeros((4, EMB), jnp.float32).at[:3].set(seg_table)
    gamma2 = gamma.reshape(1, EMB)
    beta2 = beta.reshape(1, EMB)

    xg = gathered[:n_src].reshape(b, l, EMB)
    emb = _emb_ln(xg, seg_oh, pos_table, seg_pad, gamma2, beta2)

    xp = gathered[n_src:].reshape(3 * b * t, k, EMB)
    prop_out = _prop_ln(xp, pos_table, gamma2, beta2)

    g = b * t
    pk_e = prop_out[:g].reshape(b, t, k, EMB)
    pv_e = prop_out[g:2 * g].reshape(b, t, k, EMB)
    tw_e = prop_out[2 * g:].reshape(b, t, k, EMB)
    return (emb, pk_e, pv_e, tw_e)


# speedup vs baseline: 1.1358x; 1.1229x over previous
import jax
import jax.numpy as jnp
from jax import lax
from jax.experimental import pallas as pl
from jax.experimental.pallas import tpu as pltpu
from jax.experimental.pallas import tpu_sc as plsc

EMB = 768
EPS = 1e-6
NW = 32
CHUNK = 32


NBUF = 4


def _sc_gather_body(idx_hbm, table_hbm, out_hbm, idx_v, rows_v, sem_g, sem_w):
    wid = lax.axis_index("s") * 2 + lax.axis_index("c")
    per_w = out_hbm.shape[0] // NW
    n_chunks = per_w // CHUNK
    base = wid * per_w

    descs_g = [None] * n_chunks
    descs_w = [None] * n_chunks
    for k in range(n_chunks):
        slot = k % NBUF
        if k >= NBUF:
            descs_w[k - NBUF].wait()
        off = base + k * CHUNK
        pltpu.sync_copy(idx_hbm.at[pl.ds(off, CHUNK)], idx_v.at[slot])
        descs_g[k] = pltpu.async_copy(
            table_hbm.at[idx_v.at[slot]], rows_v.at[slot], sem_g)
        if k >= 1:
            pslot = (k - 1) % NBUF
            descs_g[k - 1].wait()
            descs_w[k - 1] = pltpu.async_copy(
                rows_v.at[pslot], out_hbm.at[pl.ds(base + (k - 1) * CHUNK, CHUNK)],
                sem_w)
    k = n_chunks - 1
    descs_g[k].wait()
    descs_w[k] = pltpu.async_copy(
        rows_v.at[k % NBUF], out_hbm.at[pl.ds(base + k * CHUNK, CHUNK)], sem_w)
    for j in range(max(0, n_chunks - NBUF), n_chunks):
        descs_w[j].wait()


def _sc_gather(w_idx, word_table):
    n = w_idx.shape[0]
    return pl.kernel(
        _sc_gather_body,
        out_type=jax.ShapeDtypeStruct((n, EMB), jnp.float32),
        mesh=plsc.VectorSubcoreMesh(core_axis_name="c", subcore_axis_name="s"),
        scratch_types=[
            pltpu.VMEM((NBUF, CHUNK), jnp.int32),
            pltpu.VMEM((NBUF, CHUNK, EMB), jnp.float32),
            pltpu.SemaphoreType.DMA,
            pltpu.SemaphoreType.DMA,
        ],
    )(w_idx, word_table)


def _ln(x, g, b):
    mean = jnp.mean(x, axis=-1, keepdims=True)
    xc = x - mean
    var = jnp.mean(xc * xc, axis=-1, keepdims=True)
    return xc * lax.rsqrt(var + EPS) * g + b


def _emb_body(x_ref, oh_ref, pos_ref, segt_ref, g_ref, b_ref, o_ref):
    seg_emb = jnp.dot(oh_ref[0], segt_ref[...], preferred_element_type=jnp.float32)
    x = x_ref[0] + pos_ref[...] + seg_emb
    o_ref[0] = _ln(x, g_ref[...], b_ref[...])


def _prop_body(x_ref, pos_ref, g_ref, b_ref, o_ref):
    x = x_ref[...] + pos_ref[...][None]
    o_ref[...] = _ln(x, g_ref[...][None], b_ref[...][None])


def _emb_ln(xg, seg_oh, pos_table, seg_pad, gamma2, beta2):
    return pl.pallas_call(
        _emb_body,
        grid=(32,),
        in_specs=[
            pl.BlockSpec((1, 512, EMB), lambda i: (i, 0, 0)),
            pl.BlockSpec((1, 512, 4), lambda i: (i, 0, 0)),
            pl.BlockSpec((512, EMB), lambda i: (0, 0)),
            pl.BlockSpec((4, EMB), lambda i: (0, 0)),
            pl.BlockSpec((1, EMB), lambda i: (0, 0)),
            pl.BlockSpec((1, EMB), lambda i: (0, 0)),
        ],
        out_specs=pl.BlockSpec((1, 512, EMB), lambda i: (i, 0, 0)),
        out_shape=jax.ShapeDtypeStruct((32, 512, EMB), jnp.float32),
    )(xg, seg_oh, pos_table, seg_pad, gamma2, beta2)


def _prop_ln(xp, pos_table, gamma2, beta2):
    n = xp.shape[0]
    g = 96
    return pl.pallas_call(
        _prop_body,
        grid=(n // g,),
        in_specs=[
            pl.BlockSpec((g, 8, EMB), lambda i: (i, 0, 0)),
            pl.BlockSpec((8, EMB), lambda i: (0, 0)),
            pl.BlockSpec((1, EMB), lambda i: (0, 0)),
            pl.BlockSpec((1, EMB), lambda i: (0, 0)),
        ],
        out_specs=pl.BlockSpec((g, 8, EMB), lambda i: (i, 0, 0)),
        out_shape=jax.ShapeDtypeStruct((n, 8, EMB), jnp.float32),
    )(xp, pos_table, gamma2, beta2)


def kernel(src, seg, prop_keys, prop_values, target_words,
           word_table, pos_table, seg_table, gamma, beta):
    b, l = src.shape
    _, t, k = prop_keys.shape
    n_src = b * l
    n_prop = b * t * k

    w_idx = jnp.concatenate([
        src.reshape(-1), prop_keys.reshape(-1),
        prop_values.reshape(-1), target_words.reshape(-1),
    ]).astype(jnp.int32)

    gathered = _sc_gather(w_idx, word_table)

    seg_oh = jax.nn.one_hot(seg, 4, dtype=jnp.float32)
    seg_pad = jnp.zeros((4, EMB), jnp.float32).at[:3].set(seg_table)
    gamma2 = gamma.reshape(1, EMB)
    beta2 = beta.reshape(1, EMB)

    xg = gathered[:n_src].reshape(b, l, EMB)
    emb = _emb_ln(xg, seg_oh, pos_table, seg_pad, gamma2, beta2)

    xp = gathered[n_src:].reshape(3 * b * t, k, EMB)
    prop_out = _prop_ln(xp, pos_table, gamma2, beta2)

    g = b * t
    pk_e = prop_out[:g].reshape(b, t, k, EMB)
    pv_e = prop_out[g:2 * g].reshape(b, t, k, EMB)
    tw_e = prop_out[2 * g:].reshape(b, t, k, EMB)
    return (emb, pk_e, pv_e, tw_e)
